# initial kernel scaffold (unmeasured)
import jax
import jax.numpy as jnp
from jax import lax
from jax.experimental import pallas as pl
from jax.experimental.pallas import tpu as pltpu


def kernel(
    x,
):
    def body(*refs):
        pass

    out_shape = jax.ShapeDtypeStruct(..., jnp.float32)
    return pl.pallas_call(body, out_shape=out_shape)(...)


# baseline (device time: 164288 ns/iter reference)
import jax
import jax.numpy as jnp
from jax import lax
from jax.experimental import pallas as pl
from jax.experimental.pallas import tpu as pltpu

K = 32
BM = 128
BN = 1024

_NEG_INF = float("-inf")


def kernel(x):
    m_rows, n_loc = x.shape
    n_rb = m_rows // BM
    n_cb = n_loc // BN

    def body(x_ref, out_ref, cand_ref, recv_ref, send_sem, recv_sem):
        my_x = lax.axis_index("x")
        my_y = lax.axis_index("y")
        my_z = lax.axis_index("z")

        def row_block(rb, carry):
            rs = pl.ds(rb * BM, BM)

            def sweep(prev_max):
                m = jnp.full((BM, 1), _NEG_INF, dtype=jnp.float32)
                for cb in range(n_cb):
                    cs = pl.ds(cb * BN, BN)
                    blk = x_ref[rs, cs]
                    if prev_max is not None:
                        blk = jnp.where(blk == prev_max, _NEG_INF, blk)
                        x_ref[rs, cs] = blk
                    m = jnp.maximum(m, jnp.max(blk, axis=1, keepdims=True))
                return m

            m = sweep(None)
            cand_ref[rs, 0:1] = m
            for i in range(1, K):
                m = sweep(m)
                cand_ref[rs, i : i + 1] = m
            return carry

        lax.fori_loop(0, n_rb, row_block, 0)

        rdma = pltpu.make_async_remote_copy(
            src_ref=cand_ref,
            dst_ref=recv_ref,
            send_sem=send_sem,
            recv_sem=recv_sem,
            device_id=(1 - my_x, my_y, my_z),
            device_id_type=pl.DeviceIdType.MESH,
        )
        rdma.start()
        rdma.wait()

        def merge_block(rb, carry):
            rs = pl.ds(rb * BM, BM)
            cur = jnp.concatenate([cand_ref[rs, :], recv_ref[rs, :]], axis=1)
            m = jnp.max(cur, axis=1, keepdims=True)
            out_ref[rs, 0:1] = m
            for i in range(1, K):
                cur = jnp.where(cur == m, _NEG_INF, cur)
                m = jnp.max(cur, axis=1, keepdims=True)
                out_ref[rs, i : i + 1] = m
            return carry

        lax.fori_loop(0, n_rb, merge_block, 0)

    return pl.pallas_call(
        body,
        out_shape=jax.ShapeDtypeStruct((m_rows, K), jnp.float32),
        in_specs=[pl.BlockSpec(memory_space=pltpu.VMEM)],
        out_specs=pl.BlockSpec(memory_space=pltpu.VMEM),
        scratch_shapes=[
            pltpu.VMEM((m_rows, K), jnp.float32),
            pltpu.VMEM((m_rows, K), jnp.float32),
            pltpu.SemaphoreType.DMA,
            pltpu.SemaphoreType.DMA,
        ],
        compiler_params=pltpu.CompilerParams(vmem_limit_bytes=64 * 1024 * 1024),
    )(x)
